# baseline (speedup 1.0000x reference)
import functools

import jax
import jax.numpy as jnp
from jax import lax
from jax.experimental import pallas as pl
from jax.experimental.pallas import tpu as pltpu
from jax.experimental.pallas import tpu_sc as plsc

N_NODES = 10000
N_EDGES = 160000
D_FEAT = 256

_info = plsc.get_sparse_core_info()
NC, NS, L = _info.num_cores, _info.num_subcores, _info.num_lanes
NW = NC * NS

C = 64
NCHUNK = N_EDGES // C
ITERS = -(-NCHUNK // NW)
OUTER = (ITERS + 2) // 2


def _body(h_hbm, src_hbm, dst_hbm, out_hbm,
          iu0, iu1, iv0, iv1, ru0, ru1, rv0, rv1, sc0, sc1, sem0, sem1):
    idx_u = (iu0, iu1)
    idx_v = (iv0, iv1)
    rows_u = (ru0, ru1)
    rows_v = (rv0, rv1)
    scores = (sc0, sc1)
    sems = (sem0, sem1)

    wid = lax.axis_index("s") * NC + lax.axis_index("c")

    def start(c, b):
        @pl.when(c < NCHUNK)
        def _():
            base = pl.multiple_of(c * C, 8)
            pltpu.sync_copy(src_hbm.at[pl.ds(base, C)], idx_u[b])
            pltpu.sync_copy(dst_hbm.at[pl.ds(base, C)], idx_v[b])
            pltpu.make_async_copy(h_hbm.at[idx_u[b]], rows_u[b], sems[b]).start()
            pltpu.make_async_copy(h_hbm.at[idx_v[b]], rows_v[b], sems[b]).start()

    def finish(c, b):
        @pl.when(c < NCHUNK)
        def _():
            pltpu.make_async_copy(h_hbm.at[idx_u[b]], rows_u[b], sems[b]).wait()
            pltpu.make_async_copy(h_hbm.at[idx_v[b]], rows_v[b], sems[b]).wait()
            lane = jnp.arange(L, dtype=jnp.int32)
            for g in range(C // L):
                eids = lane + g * L

                def dstep(t, acc, _eids=eids, _b=b):
                    for u in range(8):
                        d = t * 8 + u
                        dsp = jnp.full((L,), d, dtype=jnp.int32)
                        uu = plsc.load_gather(rows_u[_b], [_eids, dsp])
                        vv = plsc.load_gather(rows_v[_b], [_eids, dsp])
                        acc = acc + uu * vv
                    return acc

                acc = lax.fori_loop(0, D_FEAT // 8, dstep,
                                    jnp.zeros((L,), jnp.float32))
                scores[b][pl.ds(g * L, L)] = acc
            base = pl.multiple_of(c * C, 8)
            pltpu.sync_copy(scores[b], out_hbm.at[pl.ds(base, C)])

    start(wid, 0)

    def outer(k, _):
        for b in (0, 1):
            i = 2 * k + b
            c = wid + i * NW
            start(wid + (i + 1) * NW, 1 - b)
            finish(c, b)
        return _

    lax.fori_loop(0, OUTER, outer, 0)


@functools.partial(
    pl.kernel,
    mesh=plsc.VectorSubcoreMesh(core_axis_name="c", subcore_axis_name="s"),
    out_type=jax.ShapeDtypeStruct((N_EDGES,), jnp.float32),
    compiler_params=pltpu.CompilerParams(
        use_tc_tiling_on_sc=False, needs_layout_passes=False),
    scratch_types=[
        pltpu.VMEM((C,), jnp.int32),
        pltpu.VMEM((C,), jnp.int32),
        pltpu.VMEM((C,), jnp.int32),
        pltpu.VMEM((C,), jnp.int32),
        pltpu.VMEM((C, D_FEAT), jnp.float32),
        pltpu.VMEM((C, D_FEAT), jnp.float32),
        pltpu.VMEM((C, D_FEAT), jnp.float32),
        pltpu.VMEM((C, D_FEAT), jnp.float32),
        pltpu.VMEM((C,), jnp.float32),
        pltpu.VMEM((C,), jnp.float32),
        pltpu.SemaphoreType.DMA,
        pltpu.SemaphoreType.DMA,
    ],
)
def _sc_dot(h_hbm, src_hbm, dst_hbm, out_hbm, *scratch):
    _body(h_hbm, src_hbm, dst_hbm, out_hbm, *scratch)


def kernel(h, edge_index):
    src = edge_index[0]
    dst = edge_index[1]
    return _sc_dot(h, src, dst)


# speedup vs baseline: 6.3809x; 6.3809x over previous
import functools

import jax
import jax.numpy as jnp
from jax import lax
from jax.experimental import pallas as pl
from jax.experimental.pallas import tpu as pltpu
from jax.experimental.pallas import tpu_sc as plsc

N_NODES = 10000
N_EDGES = 160000
D_FEAT = 256

_info = plsc.get_sparse_core_info()
NC, NS, L = _info.num_cores, _info.num_subcores, _info.num_lanes
NW = NC * NS

C = 64
NCHUNK = N_EDGES // C
ITERS = -(-NCHUNK // NW)
OUTER = (ITERS + 2) // 2


def _body(h_hbm, src_hbm, dst_hbm, out_hbm,
          iu0, iu1, iv0, iv1, ru0, ru1, rv0, rv1, sc0, sc1, sem0, sem1):
    idx_u = (iu0, iu1)
    idx_v = (iv0, iv1)
    rows_u = (ru0, ru1)
    rows_v = (rv0, rv1)
    scores = (sc0, sc1)
    sems = (sem0, sem1)

    wid = lax.axis_index("s") * NC + lax.axis_index("c")

    def start(c, b):
        @pl.when(c < NCHUNK)
        def _():
            base = pl.multiple_of(c * C, 8)
            pltpu.sync_copy(src_hbm.at[pl.ds(base, C)], idx_u[b])
            pltpu.sync_copy(dst_hbm.at[pl.ds(base, C)], idx_v[b])
            pltpu.make_async_copy(h_hbm.at[idx_u[b]], rows_u[b], sems[b]).start()
            pltpu.make_async_copy(h_hbm.at[idx_v[b]], rows_v[b], sems[b]).start()

    def finish(c, b):
        @pl.when(c < NCHUNK)
        def _():
            pltpu.make_async_copy(h_hbm.at[idx_u[b]], rows_u[b], sems[b]).wait()
            pltpu.make_async_copy(h_hbm.at[idx_v[b]], rows_v[b], sems[b]).wait()
            lane = jnp.arange(L, dtype=jnp.int32)
            for g in range(C // L):
                eids = lane + g * L

                def dstep(t, acc, _eids=eids, _b=b):
                    for u in range(8):
                        d = t * 8 + u
                        dsp = (jnp.full((L,), d, dtype=jnp.int32) + lane) & (
                            D_FEAT - 1)
                        uu = plsc.load_gather(rows_u[_b], [_eids, dsp])
                        vv = plsc.load_gather(rows_v[_b], [_eids, dsp])
                        acc = acc + uu * vv
                    return acc

                acc = lax.fori_loop(0, D_FEAT // 8, dstep,
                                    jnp.zeros((L,), jnp.float32))
                scores[b][pl.ds(g * L, L)] = acc
            base = pl.multiple_of(c * C, 8)
            pltpu.sync_copy(scores[b], out_hbm.at[pl.ds(base, C)])

    start(wid, 0)

    def outer(k, _):
        for b in (0, 1):
            i = 2 * k + b
            c = wid + i * NW
            start(wid + (i + 1) * NW, 1 - b)
            finish(c, b)
        return _

    lax.fori_loop(0, OUTER, outer, 0)


@functools.partial(
    pl.kernel,
    mesh=plsc.VectorSubcoreMesh(core_axis_name="c", subcore_axis_name="s"),
    out_type=jax.ShapeDtypeStruct((N_EDGES,), jnp.float32),
    compiler_params=pltpu.CompilerParams(
        use_tc_tiling_on_sc=False, needs_layout_passes=False),
    scratch_types=[
        pltpu.VMEM((C,), jnp.int32),
        pltpu.VMEM((C,), jnp.int32),
        pltpu.VMEM((C,), jnp.int32),
        pltpu.VMEM((C,), jnp.int32),
        pltpu.VMEM((C, D_FEAT), jnp.float32),
        pltpu.VMEM((C, D_FEAT), jnp.float32),
        pltpu.VMEM((C, D_FEAT), jnp.float32),
        pltpu.VMEM((C, D_FEAT), jnp.float32),
        pltpu.VMEM((C,), jnp.float32),
        pltpu.VMEM((C,), jnp.float32),
        pltpu.SemaphoreType.DMA,
        pltpu.SemaphoreType.DMA,
    ],
)
def _sc_dot(h_hbm, src_hbm, dst_hbm, out_hbm, *scratch):
    _body(h_hbm, src_hbm, dst_hbm, out_hbm, *scratch)


def kernel(h, edge_index):
    src = edge_index[0]
    dst = edge_index[1]
    return _sc_dot(h, src, dst)
